# baseline (speedup 1.0000x reference)
import jax
import jax.numpy as jnp
from jax.experimental import pallas as pl
from jax.experimental.pallas import tpu as pltpu

HIDDEN_DIM = 4096
NUM_EXPERTS = 8
TOP_K = 2
BLOCK_T = 1024
KSPLIT = 2
BLOCK_H = HIDDEN_DIM // KSPLIT
OUT_T = 4096


def _router_block(x_ref, w_ref, b_ref, t_ref, w_out_ref, i_out_ref, acc):
    i = pl.program_id(0)
    k = pl.program_id(1)
    x = x_ref[...]
    wt = w_ref[...].T
    part = jnp.dot(x, wt, preferred_element_type=jnp.float32)

    @pl.when(k == 0)
    def _():
        acc[...] = part

    @pl.when(k > 0)
    def _():
        acc[...] = acc[...] + part

    @pl.when(k == KSPLIT - 1)
    def _():
        logits = acc[...] + b_ref[...]
        t_safe = jnp.clip(t_ref[...], 0.1, 10.0)
        logits = logits / t_safe

        e = jax.lax.broadcasted_iota(jnp.int32, logits.shape, 1)
        m1 = jnp.max(logits, axis=1, keepdims=True)
        i1 = jnp.min(jnp.where(logits == m1, e, NUM_EXPERTS), axis=1, keepdims=True)
        masked = jnp.where(e == i1, -jnp.inf, logits)
        m2 = jnp.max(masked, axis=1, keepdims=True)
        i2 = jnp.min(jnp.where(masked == m2, e, NUM_EXPERTS), axis=1, keepdims=True)

        t = jnp.exp(m2 - m1)
        denom = 1.0 + t
        base = (i % (OUT_T // BLOCK_T)) * BLOCK_T
        w_out_ref[pl.ds(base, BLOCK_T), :] = jnp.concatenate(
            [1.0 / denom, t / denom], axis=1)
        i_out_ref[pl.ds(base, BLOCK_T), :] = jnp.concatenate([i1, i2], axis=1)


def kernel(hidden_states, pressure_bias, temperature_field, W):
    b, s, h = hidden_states.shape
    n_tok = b * s
    x = hidden_states.reshape(n_tok, h)
    bias = pressure_bias.reshape(1, NUM_EXPERTS)
    temp = temperature_field.reshape(1, NUM_EXPERTS)

    grid = (n_tok // BLOCK_T, KSPLIT)
    w_out, i_out = pl.pallas_call(
        _router_block,
        grid=grid,
        in_specs=[
            pl.BlockSpec((BLOCK_T, BLOCK_H), lambda i, k: (i, k)),
            pl.BlockSpec((NUM_EXPERTS, BLOCK_H), lambda i, k: (0, k)),
            pl.BlockSpec((1, NUM_EXPERTS), lambda i, k: (0, 0)),
            pl.BlockSpec((1, NUM_EXPERTS), lambda i, k: (0, 0)),
        ],
        out_specs=[
            pl.BlockSpec((OUT_T, TOP_K), lambda i, k: (i // (OUT_T // BLOCK_T), 0)),
            pl.BlockSpec((OUT_T, TOP_K), lambda i, k: (i // (OUT_T // BLOCK_T), 0)),
        ],
        out_shape=[
            jax.ShapeDtypeStruct((n_tok, TOP_K), jnp.float32),
            jax.ShapeDtypeStruct((n_tok, TOP_K), jnp.int32),
        ],
        scratch_shapes=[
            pltpu.MemorySpace.VMEM((BLOCK_T, NUM_EXPERTS), jnp.float32),
        ],
        compiler_params=pltpu.CompilerParams(
            vmem_limit_bytes=100 * 1024 * 1024,
        ),
    )(x, W, bias, temp)

    return (w_out.reshape(b, s, TOP_K), i_out.reshape(b, s, TOP_K))


# speedup vs baseline: 1.1386x; 1.1386x over previous
import jax
import jax.numpy as jnp
from jax.experimental import pallas as pl
from jax.experimental.pallas import tpu as pltpu

HIDDEN_DIM = 4096
NUM_EXPERTS = 8
TOP_K = 2
BLOCK_T = 1024
OUT_T = 4096


def _router_block(x_ref, w_ref, b_ref, t_ref, w_out_ref, i_out_ref):
    i = pl.program_id(0)
    x = x_ref[...]
    wt = w_ref[...].T
    logits = jnp.dot(x, wt, preferred_element_type=jnp.float32)
    logits = logits + b_ref[...]
    t_safe = jnp.clip(t_ref[...], 0.1, 10.0)
    logits = logits / t_safe

    e = jax.lax.broadcasted_iota(jnp.int32, logits.shape, 1)
    m1 = jnp.max(logits, axis=1, keepdims=True)
    i1 = jnp.min(jnp.where(logits == m1, e, NUM_EXPERTS), axis=1, keepdims=True)
    masked = jnp.where(e == i1, -jnp.inf, logits)
    m2 = jnp.max(masked, axis=1, keepdims=True)
    i2 = jnp.min(jnp.where(masked == m2, e, NUM_EXPERTS), axis=1, keepdims=True)

    t = jnp.exp(m2 - m1)
    denom = 1.0 + t
    base = (i % (OUT_T // BLOCK_T)) * BLOCK_T
    w_out_ref[pl.ds(base, BLOCK_T), :] = jnp.concatenate(
        [1.0 / denom, t / denom], axis=1)
    i_out_ref[pl.ds(base, BLOCK_T), :] = jnp.concatenate([i1, i2], axis=1)


def kernel(hidden_states, pressure_bias, temperature_field, W):
    b, s, h = hidden_states.shape
    n_tok = b * s
    x = hidden_states.reshape(n_tok, h)
    bias = pressure_bias.reshape(1, NUM_EXPERTS)
    temp = temperature_field.reshape(1, NUM_EXPERTS)

    grid = (n_tok // BLOCK_T,)
    w_out, i_out = pl.pallas_call(
        _router_block,
        grid=grid,
        in_specs=[
            pl.BlockSpec((BLOCK_T, h), lambda i: (i, 0)),
            pl.BlockSpec((NUM_EXPERTS, h), lambda i: (0, 0)),
            pl.BlockSpec((1, NUM_EXPERTS), lambda i: (0, 0)),
            pl.BlockSpec((1, NUM_EXPERTS), lambda i: (0, 0)),
        ],
        out_specs=[
            pl.BlockSpec((OUT_T, TOP_K), lambda i: (i // (OUT_T // BLOCK_T), 0)),
            pl.BlockSpec((OUT_T, TOP_K), lambda i: (i // (OUT_T // BLOCK_T), 0)),
        ],
        out_shape=[
            jax.ShapeDtypeStruct((n_tok, TOP_K), jnp.float32),
            jax.ShapeDtypeStruct((n_tok, TOP_K), jnp.int32),
        ],
        compiler_params=pltpu.CompilerParams(
            vmem_limit_bytes=100 * 1024 * 1024,
        ),
    )(x, W, bias, temp)

    return (w_out.reshape(b, s, TOP_K), i_out.reshape(b, s, TOP_K))


# speedup vs baseline: 1.1457x; 1.0062x over previous
import jax
import jax.numpy as jnp
from jax.experimental import pallas as pl
from jax.experimental.pallas import tpu as pltpu

HIDDEN_DIM = 4096
NUM_EXPERTS = 8
TOP_K = 2
BLOCK_T = 1024


def _router_block(x_ref, w_ref, b_ref, t_ref, w_out_ref, i_out_ref):
    x = x_ref[...]
    wt = w_ref[...].T
    logits = jnp.dot(x, wt, preferred_element_type=jnp.float32)
    logits = logits + b_ref[...]
    t_safe = jnp.clip(t_ref[...], 0.1, 10.0)
    logits = logits / t_safe

    e = jax.lax.broadcasted_iota(jnp.int32, logits.shape, 1)
    m1 = jnp.max(logits, axis=1, keepdims=True)
    i1 = jnp.min(jnp.where(logits == m1, e, NUM_EXPERTS), axis=1, keepdims=True)
    masked = jnp.where(e == i1, -jnp.inf, logits)
    m2 = jnp.max(masked, axis=1, keepdims=True)
    i2 = jnp.min(jnp.where(masked == m2, e, NUM_EXPERTS), axis=1, keepdims=True)

    t = jnp.exp(m2 - m1)
    denom = 1.0 + t
    w_out_ref[...] = jnp.concatenate([1.0 / denom, t / denom], axis=1)
    i_out_ref[...] = jnp.concatenate([i1, i2], axis=1)


def kernel(hidden_states, pressure_bias, temperature_field, W):
    b, s, h = hidden_states.shape
    n_tok = b * s
    x = hidden_states.reshape(n_tok, h)
    bias = pressure_bias.reshape(1, NUM_EXPERTS)
    temp = temperature_field.reshape(1, NUM_EXPERTS)

    grid = (n_tok // BLOCK_T,)
    w_out, i_out = pl.pallas_call(
        _router_block,
        grid=grid,
        in_specs=[
            pl.BlockSpec((BLOCK_T, h), lambda i: (i, 0)),
            pl.BlockSpec((NUM_EXPERTS, h), lambda i: (0, 0)),
            pl.BlockSpec((1, NUM_EXPERTS), lambda i: (0, 0)),
            pl.BlockSpec((1, NUM_EXPERTS), lambda i: (0, 0)),
        ],
        out_specs=[
            pl.BlockSpec((BLOCK_T, TOP_K), lambda i: (i, 0)),
            pl.BlockSpec((BLOCK_T, TOP_K), lambda i: (i, 0)),
        ],
        out_shape=[
            jax.ShapeDtypeStruct((n_tok, TOP_K), jnp.float32),
            jax.ShapeDtypeStruct((n_tok, TOP_K), jnp.int32),
        ],
        compiler_params=pltpu.CompilerParams(
            vmem_limit_bytes=100 * 1024 * 1024,
        ),
    )(x, W, bias, temp)

    return (w_out.reshape(b, s, TOP_K), i_out.reshape(b, s, TOP_K))
